# Initial kernel scaffold; baseline (speedup 1.0000x reference)
import jax
import jax.numpy as jnp
from jax.experimental import pallas as pl


def kernel(prediction, target):
    raise NotImplementedError("write your pallas kernel here")


# speedup vs baseline: 2.5570x; 2.5570x over previous
import functools
import math

import jax
import jax.numpy as jnp
from jax import lax
from jax.experimental import pallas as pl
from jax.experimental.pallas import tpu as pltpu
from jax.experimental.pallas import tpu_sc as plsc

SIZE = 32000
PADDING_INDEX = 0
SMOOTHING = 0.1
CONFIDENCE = 1.0 - SMOOTHING
EPS = SMOOTHING / (SIZE - 2)
ROW_CONST = SMOOTHING * math.log(EPS) + CONFIDENCE * math.log(CONFIDENCE)

N_ROWS = 8192

BR = 512
BC = 3200
RT = N_ROWS // BR
CT = SIZE // BC


def _tc_body(x_ref, w_ref, out_ref):
    i = pl.program_id(0)
    j = pl.program_id(1)

    @pl.when((i == 0) & (j == 0))
    def _():
        out_ref[0, 0] = 0.0

    x = x_ref[...]
    w = w_ref[:, 0]
    rowsum = jnp.sum(x, axis=1)
    s = jnp.sum(rowsum * w)
    corr = jnp.sum(x[:, 0] * w)
    s = s - jnp.where(j == 0, corr, 0.0)
    out_ref[0, 0] += s


def _tc_masked_sum(prediction, valid2):
    return pl.pallas_call(
        _tc_body,
        grid=(RT, CT),
        in_specs=[
            pl.BlockSpec((BR, BC), lambda i, j: (i, j)),
            pl.BlockSpec((BR, 1), lambda i, j: (i, 0)),
        ],
        out_specs=pl.BlockSpec(memory_space=pltpu.SMEM),
        out_shape=jax.ShapeDtypeStruct((1, 1), jnp.float32),
    )(prediction, valid2)


NC = 2
NS = 16
NW = NC * NS
RPW = N_ROWS // NW
CHUNK = 128
NCHUNK = RPW // CHUNK


@functools.partial(
    pl.kernel,
    mesh=plsc.VectorSubcoreMesh(core_axis_name="c", subcore_axis_name="s"),
    out_type=[
        jax.ShapeDtypeStruct((NW, 16), jnp.float32),
        jax.ShapeDtypeStruct((NW, 16), jnp.float32),
    ],
    scratch_types=[
        pltpu.VMEM((RPW,), jnp.int32),
        pltpu.VMEM((CHUNK,), jnp.int32),
        pltpu.VMEM((CHUNK,), jnp.float32),
        pltpu.VMEM((16,), jnp.float32),
        pltpu.VMEM((16,), jnp.float32),
        pltpu.SemaphoreType.DMA,
    ],
)
def _sc_gather(pred_flat_hbm, tgt_hbm, sums_hbm, cnts_hbm,
               tgt_v, idx_v, val_v, stage_s, stage_c, sem):
    cid = lax.axis_index("c")
    sid = lax.axis_index("s")
    wid = sid * NC + cid
    base = pl.multiple_of(wid * RPW, RPW)

    pltpu.sync_copy(tgt_hbm.at[pl.ds(base, RPW)], tgt_v)

    acc = jnp.zeros((16,), jnp.float32)
    cnt = jnp.zeros((16,), jnp.float32)
    for c in range(NCHUNK):
        for k in range(CHUNK // 16):
            t = tgt_v[pl.ds(c * CHUNK + k * 16, 16)]
            rows = base + c * CHUNK + k * 16 + lax.iota(jnp.int32, 16)
            idx_v[pl.ds(k * 16, 16)] = rows * SIZE + t
        pltpu.async_copy(pred_flat_hbm.at[idx_v], val_v, sem).wait()
        for k in range(CHUNK // 16):
            v = val_v[pl.ds(k * 16, 16)]
            t = tgt_v[pl.ds(c * CHUNK + k * 16, 16)]
            m = t != PADDING_INDEX
            acc = acc + jnp.where(m, v, 0.0)
            cnt = cnt + jnp.where(m, 1.0, 0.0)

    stage_s[...] = acc
    stage_c[...] = cnt
    pltpu.sync_copy(stage_s, sums_hbm.at[wid])
    pltpu.sync_copy(stage_c, cnts_hbm.at[wid])


def kernel(prediction, target):
    target = target.astype(jnp.int32)
    valid2 = (target != PADDING_INDEX).astype(jnp.float32)[:, None]

    a = _tc_masked_sum(prediction, valid2)[0, 0]

    pred_flat = jnp.reshape(prediction, (-1,))
    sums, cnts = _sc_gather(pred_flat, target)
    t3 = jnp.sum(sums)
    n = jnp.sum(cnts)

    return n * ROW_CONST - EPS * a - (CONFIDENCE - EPS) * t3


# speedup vs baseline: 2.5785x; 1.0084x over previous
import functools
import math

import jax
import jax.numpy as jnp
from jax import lax
from jax.experimental import pallas as pl
from jax.experimental.pallas import tpu as pltpu
from jax.experimental.pallas import tpu_sc as plsc

SIZE = 32000
PADDING_INDEX = 0
SMOOTHING = 0.1
CONFIDENCE = 1.0 - SMOOTHING
EPS = SMOOTHING / (SIZE - 2)
ROW_CONST = SMOOTHING * math.log(EPS) + CONFIDENCE * math.log(CONFIDENCE)

N_ROWS = 8192

BR = 128
RT = N_ROWS // BR


def _tc_body(x_ref, w_ref, out_ref):
    i = pl.program_id(0)

    @pl.when(i == 0)
    def _():
        out_ref[0, 0] = 0.0

    x = x_ref[...]
    w = w_ref[:, 0]
    rowsum = jnp.sum(x, axis=1)
    out_ref[0, 0] += jnp.sum((rowsum - x[:, 0]) * w)


def _tc_masked_sum(prediction, valid2):
    return pl.pallas_call(
        _tc_body,
        grid=(RT,),
        in_specs=[
            pl.BlockSpec((BR, SIZE), lambda i: (i, 0)),
            pl.BlockSpec((BR, 1), lambda i: (i, 0)),
        ],
        out_specs=pl.BlockSpec(memory_space=pltpu.SMEM),
        out_shape=jax.ShapeDtypeStruct((1, 1), jnp.float32),
    )(prediction, valid2)


NC = 2
NS = 16
NW = NC * NS
RPW = N_ROWS // NW
CHUNK = 128
NCHUNK = RPW // CHUNK


@functools.partial(
    pl.kernel,
    mesh=plsc.VectorSubcoreMesh(core_axis_name="c", subcore_axis_name="s"),
    out_type=[
        jax.ShapeDtypeStruct((NW, 16), jnp.float32),
        jax.ShapeDtypeStruct((NW, 16), jnp.float32),
    ],
    scratch_types=[
        pltpu.VMEM((RPW,), jnp.int32),
        pltpu.VMEM((CHUNK,), jnp.int32),
        pltpu.VMEM((CHUNK,), jnp.float32),
        pltpu.VMEM((16,), jnp.float32),
        pltpu.VMEM((16,), jnp.float32),
        pltpu.SemaphoreType.DMA,
    ],
)
def _sc_gather(pred_flat_hbm, tgt_hbm, sums_hbm, cnts_hbm,
               tgt_v, idx_v, val_v, stage_s, stage_c, sem):
    cid = lax.axis_index("c")
    sid = lax.axis_index("s")
    wid = sid * NC + cid
    base = pl.multiple_of(wid * RPW, RPW)

    pltpu.sync_copy(tgt_hbm.at[pl.ds(base, RPW)], tgt_v)

    acc = jnp.zeros((16,), jnp.float32)
    cnt = jnp.zeros((16,), jnp.float32)
    for c in range(NCHUNK):
        for k in range(CHUNK // 16):
            t = tgt_v[pl.ds(c * CHUNK + k * 16, 16)]
            rows = base + c * CHUNK + k * 16 + lax.iota(jnp.int32, 16)
            idx_v[pl.ds(k * 16, 16)] = rows * SIZE + t
        pltpu.async_copy(pred_flat_hbm.at[idx_v], val_v, sem).wait()
        for k in range(CHUNK // 16):
            v = val_v[pl.ds(k * 16, 16)]
            t = tgt_v[pl.ds(c * CHUNK + k * 16, 16)]
            m = t != PADDING_INDEX
            acc = acc + jnp.where(m, v, 0.0)
            cnt = cnt + jnp.where(m, 1.0, 0.0)

    stage_s[...] = acc
    stage_c[...] = cnt
    pltpu.sync_copy(stage_s, sums_hbm.at[wid])
    pltpu.sync_copy(stage_c, cnts_hbm.at[wid])


def kernel(prediction, target):
    target = target.astype(jnp.int32)
    valid2 = (target != PADDING_INDEX).astype(jnp.float32)[:, None]

    a = _tc_masked_sum(prediction, valid2)[0, 0]

    pred_flat = jnp.reshape(prediction, (-1,))
    sums, cnts = _sc_gather(pred_flat, target)
    t3 = jnp.sum(sums)
    n = jnp.sum(cnts)

    return n * ROW_CONST - EPS * a - (CONFIDENCE - EPS) * t3
